# baseline (speedup 1.0000x reference)
import functools

import jax
import jax.numpy as jnp
from jax import lax
from jax.experimental import pallas as pl
from jax.experimental.pallas import tpu as pltpu
from jax.experimental.pallas import tpu_sc as plsc

TOPK = 2
BM = 256


def _routing_indices(sel, weights, T, E, NT, NP):
    P = T * TOPK
    e_flat = sel.reshape(-1).astype(jnp.int32)
    w_flat = weights.reshape(-1)
    onehot = (e_flat[:, None] == jnp.arange(E, dtype=jnp.int32)[None, :]
              ).astype(jnp.int32)
    ranks_incl = jnp.cumsum(onehot, axis=0)
    rank = jnp.sum(onehot * ranks_incl, axis=1) - 1
    counts = ranks_incl[-1]
    padded_counts = ((counts + BM - 1) // BM) * BM
    padded_start = jnp.concatenate(
        [jnp.zeros((1,), jnp.int32), jnp.cumsum(padded_counts)[:-1]])
    start_of_pair = jnp.sum(onehot * padded_start[None, :], axis=1)
    ppos = start_of_pair + rank
    tok_of_pair = jnp.arange(P, dtype=jnp.int32) // TOPK
    src_token = jnp.zeros((NP,), jnp.int32).at[ppos].set(tok_of_pair)
    coeff = jnp.zeros((NP,), jnp.float32).at[ppos].set(w_flat)
    tile_row0 = jnp.arange(NT, dtype=jnp.int32) * BM
    tile_expert = jnp.clip(
        (tile_row0[:, None] >= padded_start[None, :]).astype(jnp.int32)
        .sum(axis=1) - 1, 0, E - 1).astype(jnp.int32)
    return src_token, coeff, tile_expert, ppos


def _sc_gather_rows(table, idx, n_rows):
    H = table.shape[1]
    info = plsc.get_sparse_core_info()
    NC, NS = info.num_cores, info.num_subcores
    NW = NC * NS
    rows_pw = n_rows // NW
    ch = 32
    while rows_pw % ch:
        ch //= 2
    nch = rows_pw // ch
    mesh = plsc.VectorSubcoreMesh(core_axis_name="c", subcore_axis_name="s")

    @functools.partial(
        pl.kernel, mesh=mesh,
        out_type=jax.ShapeDtypeStruct((n_rows, H), jnp.float32),
        scratch_types=[
            pltpu.VMEM((rows_pw,), jnp.int32),
            pltpu.VMEM((ch, H), jnp.float32),
            pltpu.VMEM((ch, H), jnp.float32),
            pltpu.SemaphoreType.DMA,
            pltpu.SemaphoreType.DMA,
            pltpu.SemaphoreType.DMA,
            pltpu.SemaphoreType.DMA,
        ],
    )
    def k(idx_hbm, tab_hbm, out_hbm, idx_v, r0, r1, gs0, gs1, os0, os1):
        wid = lax.axis_index("s") * NC + lax.axis_index("c")
        rows_v = [r0, r1]
        gsem = [gs0, gs1]
        osem = [os0, os1]

        def base(c):
            return wid * rows_pw + c * ch

        pltpu.sync_copy(idx_hbm.at[pl.ds(wid * rows_pw, rows_pw)], idx_v)

        gather = [None] * nch
        out = [None] * nch
        gather[0] = pltpu.async_copy(
            tab_hbm.at[idx_v.at[pl.ds(0, ch)]], rows_v[0], gsem[0])
        for c in range(nch):
            b = c % 2
            nb_ = (c + 1) % 2
            if c + 1 < nch:
                if c - 1 >= 0:
                    out[c - 1].wait()
                gather[c + 1] = pltpu.async_copy(
                    tab_hbm.at[idx_v.at[pl.ds((c + 1) * ch, ch)]],
                    rows_v[nb_], gsem[nb_])
            gather[c].wait()
            out[c] = pltpu.async_copy(
                rows_v[b], out_hbm.at[pl.ds(base(c), ch)], osem[b])
        if nch >= 2:
            out[nch - 2].wait()
        out[nch - 1].wait()

    return k(idx, table)


def _sc_combine(og, p0, p1, T):
    H = og.shape[1]
    info = plsc.get_sparse_core_info()
    NC, NS = info.num_cores, info.num_subcores
    NW = NC * NS
    tok_pw = T // NW
    ch = 32
    while tok_pw % ch:
        ch //= 2
    nch = tok_pw // ch
    mesh = plsc.VectorSubcoreMesh(core_axis_name="c", subcore_axis_name="s")

    @functools.partial(
        pl.kernel, mesh=mesh,
        out_type=jax.ShapeDtypeStruct((T, H), jnp.float32),
        scratch_types=[
            pltpu.VMEM((ch,), jnp.int32),
            pltpu.VMEM((ch,), jnp.int32),
            pltpu.VMEM((ch, H), jnp.float32),
            pltpu.VMEM((ch, H), jnp.float32),
            pltpu.SemaphoreType.DMA,
        ],
    )
    def k(p0_hbm, p1_hbm, og_hbm, out_hbm, i0_v, i1_v, r0_v, r1_v, sem):
        wid = lax.axis_index("s") * NC + lax.axis_index("c")

        def body(c, carry):
            base = wid * tok_pw + c * ch
            pltpu.sync_copy(p0_hbm.at[pl.ds(base, ch)], i0_v)
            pltpu.sync_copy(p1_hbm.at[pl.ds(base, ch)], i1_v)
            pltpu.async_copy(og_hbm.at[i0_v], r0_v, sem).wait()
            pltpu.async_copy(og_hbm.at[i1_v], r1_v, sem).wait()

            def add_row(r, c2):
                for j in range(H // 16):
                    sl = pl.ds(j * 16, 16)
                    r0_v[r, sl] = r0_v[r, sl] + r1_v[r, sl]
                return c2

            lax.fori_loop(0, ch, add_row, 0)
            pltpu.sync_copy(r0_v, out_hbm.at[pl.ds(base, ch)])
            return carry

        lax.fori_loop(0, nch, body, 0)

    return k(p0, p1, og)


def kernel(x, gate_w, w1, w2, w3):
    T, H = x.shape
    E = gate_w.shape[0]
    EF = w1.shape[0]
    FFN = EF // E
    P = T * TOPK
    NT = P // BM + E
    NP = NT * BM

    gate_logits = x @ gate_w.T
    all_probs = jax.nn.softmax(gate_logits.astype(jnp.float32), axis=1)
    m1 = jnp.max(all_probs, axis=1)
    e1 = jnp.argmax(all_probs, axis=1).astype(jnp.int32)
    masked = jnp.where(
        jnp.arange(E, dtype=jnp.int32)[None, :] == e1[:, None],
        -jnp.inf, all_probs)
    m2 = jnp.max(masked, axis=1)
    e2 = jnp.argmax(masked, axis=1).astype(jnp.int32)
    sel = jnp.stack([e1, e2], axis=1)
    topk_vals = jnp.stack([m1, m2], axis=1)
    weights = topk_vals / jnp.sum(topk_vals, axis=-1, keepdims=True)

    src_token, coeff, tile_expert, ppos = _routing_indices(sel, weights, T, E, NT, NP)

    xs = _sc_gather_rows(x, src_token, NP)

    nb = 2
    FH = FFN // nb
    coeff2 = coeff[:, None]

    def body(te_ref, xs_ref, w1_ref, w3_ref, w2_ref, c_ref, *rest):
        if len(rest) == 2:
            oa_ref, o_ref = rest
        else:
            (o_ref,) = rest
            oa_ref = None
        xb = xs_ref[...].astype(jnp.bfloat16)
        fc = FH
        acc = jnp.zeros((xs_ref.shape[0], H), jnp.float32)
        for q in range(FH // fc):
            w1b = w1_ref[pl.ds(q * fc, fc), :].astype(jnp.bfloat16)
            w3b = w3_ref[pl.ds(q * fc, fc), :].astype(jnp.bfloat16)
            w2b = w2_ref[pl.ds(q * fc, fc), :].astype(jnp.bfloat16)
            a = lax.dot_general(xb, w1b, (((1,), (1,)), ((), ())),
                                preferred_element_type=jnp.float32)
            b = lax.dot_general(xb, w3b, (((1,), (1,)), ((), ())),
                                preferred_element_type=jnp.float32)
            h = ((a * jax.lax.logistic(a)) * b).astype(jnp.bfloat16)
            acc = acc + lax.dot_general(h, w2b, (((1,), (0,)), ((), ())),
                                        preferred_element_type=jnp.float32)
        o = acc * c_ref[...]
        if oa_ref is not None:
            o = o + oa_ref[...]
        o_ref[...] = o

    def half_call(f, prev):
        in_specs = [
            pl.BlockSpec((BM, H), lambda i, te: (i, 0)),
            pl.BlockSpec((FH, H), lambda i, te: (te[i] * nb + f, 0)),
            pl.BlockSpec((FH, H), lambda i, te: (te[i] * nb + f, 0)),
            pl.BlockSpec((FH, H), lambda i, te: (te[i] * nb + f, 0)),
            pl.BlockSpec((BM, 1), lambda i, te: (i, 0)),
        ]
        args = [tile_expert, xs, w1, w3, w2, coeff2]
        if prev is not None:
            in_specs.append(pl.BlockSpec((BM, H), lambda i, te: (i, 0)))
            args.append(prev)
        grid_spec = pltpu.PrefetchScalarGridSpec(
            num_scalar_prefetch=1,
            grid=(NT,),
            in_specs=in_specs,
            out_specs=pl.BlockSpec((BM, H), lambda i, te: (i, 0)),
        )
        return pl.pallas_call(
            body,
            grid_spec=grid_spec,
            out_shape=jax.ShapeDtypeStruct((NP, H), jnp.float32),
            compiler_params=pltpu.CompilerParams(
                dimension_semantics=("arbitrary",)),
        )(*args)

    oa = half_call(0, None)
    og = half_call(1, oa)

    p0 = ppos[0::TOPK]
    p1 = ppos[1::TOPK]
    return _sc_combine(og, p0, p1, T)


# speedup vs baseline: 1.0025x; 1.0025x over previous
import functools

import jax
import jax.numpy as jnp
from jax import lax
from jax.experimental import pallas as pl
from jax.experimental.pallas import tpu as pltpu
from jax.experimental.pallas import tpu_sc as plsc

TOPK = 2
BM = 256


def _routing_indices(sel, weights, T, E, NT, NP):
    P = T * TOPK
    e_flat = sel.reshape(-1).astype(jnp.int32)
    w_flat = weights.reshape(-1)
    onehot = (e_flat[:, None] == jnp.arange(E, dtype=jnp.int32)[None, :]
              ).astype(jnp.int32)
    ranks_incl = jnp.cumsum(onehot, axis=0)
    rank = jnp.sum(onehot * ranks_incl, axis=1) - 1
    counts = ranks_incl[-1]
    padded_counts = ((counts + BM - 1) // BM) * BM
    padded_start = jnp.concatenate(
        [jnp.zeros((1,), jnp.int32), jnp.cumsum(padded_counts)[:-1]])
    start_of_pair = jnp.sum(onehot * padded_start[None, :], axis=1)
    ppos = start_of_pair + rank
    tok_of_pair = jnp.arange(P, dtype=jnp.int32) // TOPK
    packed = jnp.zeros((NP, 2), jnp.float32).at[ppos].set(
        jnp.stack([tok_of_pair.astype(jnp.float32), w_flat], axis=1))
    src_token = packed[:, 0].astype(jnp.int32)
    coeff = packed[:, 1]
    tile_row0 = jnp.arange(NT, dtype=jnp.int32) * BM
    tile_expert = jnp.clip(
        (tile_row0[:, None] >= padded_start[None, :]).astype(jnp.int32)
        .sum(axis=1) - 1, 0, E - 1).astype(jnp.int32)
    return src_token, coeff, tile_expert, ppos


def _sc_gather_rows(table, idx, n_rows):
    H = table.shape[1]
    info = plsc.get_sparse_core_info()
    NC, NS = info.num_cores, info.num_subcores
    NW = NC * NS
    rows_pw = n_rows // NW
    ch = 32
    while rows_pw % ch:
        ch //= 2
    nch = rows_pw // ch
    mesh = plsc.VectorSubcoreMesh(core_axis_name="c", subcore_axis_name="s")

    @functools.partial(
        pl.kernel, mesh=mesh,
        out_type=jax.ShapeDtypeStruct((n_rows, H), jnp.float32),
        scratch_types=[
            pltpu.VMEM((rows_pw,), jnp.int32),
            pltpu.VMEM((ch, H), jnp.float32),
            pltpu.VMEM((ch, H), jnp.float32),
            pltpu.SemaphoreType.DMA,
            pltpu.SemaphoreType.DMA,
            pltpu.SemaphoreType.DMA,
            pltpu.SemaphoreType.DMA,
        ],
    )
    def k(idx_hbm, tab_hbm, out_hbm, idx_v, r0, r1, gs0, gs1, os0, os1):
        wid = lax.axis_index("s") * NC + lax.axis_index("c")
        rows_v = [r0, r1]
        gsem = [gs0, gs1]
        osem = [os0, os1]

        def base(c):
            return wid * rows_pw + c * ch

        pltpu.sync_copy(idx_hbm.at[pl.ds(wid * rows_pw, rows_pw)], idx_v)

        gather = [None] * nch
        out = [None] * nch
        gather[0] = pltpu.async_copy(
            tab_hbm.at[idx_v.at[pl.ds(0, ch)]], rows_v[0], gsem[0])
        for c in range(nch):
            b = c % 2
            nb_ = (c + 1) % 2
            if c + 1 < nch:
                if c - 1 >= 0:
                    out[c - 1].wait()
                gather[c + 1] = pltpu.async_copy(
                    tab_hbm.at[idx_v.at[pl.ds((c + 1) * ch, ch)]],
                    rows_v[nb_], gsem[nb_])
            gather[c].wait()
            out[c] = pltpu.async_copy(
                rows_v[b], out_hbm.at[pl.ds(base(c), ch)], osem[b])
        if nch >= 2:
            out[nch - 2].wait()
        out[nch - 1].wait()

    return k(idx, table)


def _sc_combine(og, p0, p1, T):
    H = og.shape[1]
    info = plsc.get_sparse_core_info()
    NC, NS = info.num_cores, info.num_subcores
    NW = NC * NS
    tok_pw = T // NW
    ch = 32
    while tok_pw % ch:
        ch //= 2
    nch = tok_pw // ch
    mesh = plsc.VectorSubcoreMesh(core_axis_name="c", subcore_axis_name="s")

    @functools.partial(
        pl.kernel, mesh=mesh,
        out_type=jax.ShapeDtypeStruct((T, H), jnp.float32),
        scratch_types=[
            pltpu.VMEM((ch,), jnp.int32),
            pltpu.VMEM((ch,), jnp.int32),
            pltpu.VMEM((ch, H), jnp.float32),
            pltpu.VMEM((ch, H), jnp.float32),
            pltpu.SemaphoreType.DMA,
        ],
    )
    def k(p0_hbm, p1_hbm, og_hbm, out_hbm, i0_v, i1_v, r0_v, r1_v, sem):
        wid = lax.axis_index("s") * NC + lax.axis_index("c")

        def body(c, carry):
            base = wid * tok_pw + c * ch
            pltpu.sync_copy(p0_hbm.at[pl.ds(base, ch)], i0_v)
            pltpu.sync_copy(p1_hbm.at[pl.ds(base, ch)], i1_v)
            pltpu.async_copy(og_hbm.at[i0_v], r0_v, sem).wait()
            pltpu.async_copy(og_hbm.at[i1_v], r1_v, sem).wait()

            def add_row(r, c2):
                for j in range(H // 16):
                    sl = pl.ds(j * 16, 16)
                    r0_v[r, sl] = r0_v[r, sl] + r1_v[r, sl]
                return c2

            lax.fori_loop(0, ch, add_row, 0)
            pltpu.sync_copy(r0_v, out_hbm.at[pl.ds(base, ch)])
            return carry

        lax.fori_loop(0, nch, body, 0)

    return k(p0, p1, og)


def kernel(x, gate_w, w1, w2, w3):
    T, H = x.shape
    E = gate_w.shape[0]
    EF = w1.shape[0]
    FFN = EF // E
    P = T * TOPK
    NT = P // BM + E
    NP = NT * BM

    gate_logits = x @ gate_w.T
    all_probs = jax.nn.softmax(gate_logits.astype(jnp.float32), axis=1)
    m1 = jnp.max(all_probs, axis=1)
    e1 = jnp.argmax(all_probs, axis=1).astype(jnp.int32)
    masked = jnp.where(
        jnp.arange(E, dtype=jnp.int32)[None, :] == e1[:, None],
        -jnp.inf, all_probs)
    m2 = jnp.max(masked, axis=1)
    e2 = jnp.argmax(masked, axis=1).astype(jnp.int32)
    sel = jnp.stack([e1, e2], axis=1)
    topk_vals = jnp.stack([m1, m2], axis=1)
    weights = topk_vals / jnp.sum(topk_vals, axis=-1, keepdims=True)

    src_token, coeff, tile_expert, ppos = _routing_indices(sel, weights, T, E, NT, NP)

    xs = _sc_gather_rows(x, src_token, NP)

    nb = 2
    FH = FFN // nb
    coeff2 = coeff[:, None]

    def body(te_ref, xs_ref, w1_ref, w3_ref, w2_ref, c_ref, *rest):
        if len(rest) == 2:
            oa_ref, o_ref = rest
        else:
            (o_ref,) = rest
            oa_ref = None
        xb = xs_ref[...].astype(jnp.bfloat16)
        fc = FH
        acc = jnp.zeros((xs_ref.shape[0], H), jnp.float32)
        for q in range(FH // fc):
            w1b = w1_ref[pl.ds(q * fc, fc), :].astype(jnp.bfloat16)
            w3b = w3_ref[pl.ds(q * fc, fc), :].astype(jnp.bfloat16)
            w2b = w2_ref[pl.ds(q * fc, fc), :].astype(jnp.bfloat16)
            a = lax.dot_general(xb, w1b, (((1,), (1,)), ((), ())),
                                preferred_element_type=jnp.float32)
            b = lax.dot_general(xb, w3b, (((1,), (1,)), ((), ())),
                                preferred_element_type=jnp.float32)
            h = ((a * jax.lax.logistic(a)) * b).astype(jnp.bfloat16)
            acc = acc + lax.dot_general(h, w2b, (((1,), (0,)), ((), ())),
                                        preferred_element_type=jnp.float32)
        o = acc * c_ref[...]
        if oa_ref is not None:
            o = o + oa_ref[...]
        o_ref[...] = o

    def half_call(f, prev):
        in_specs = [
            pl.BlockSpec((BM, H), lambda i, te: (i, 0)),
            pl.BlockSpec((FH, H), lambda i, te: (te[i] * nb + f, 0)),
            pl.BlockSpec((FH, H), lambda i, te: (te[i] * nb + f, 0)),
            pl.BlockSpec((FH, H), lambda i, te: (te[i] * nb + f, 0)),
            pl.BlockSpec((BM, 1), lambda i, te: (i, 0)),
        ]
        args = [tile_expert, xs, w1, w3, w2, coeff2]
        if prev is not None:
            in_specs.append(pl.BlockSpec((BM, H), lambda i, te: (i, 0)))
            args.append(prev)
        grid_spec = pltpu.PrefetchScalarGridSpec(
            num_scalar_prefetch=1,
            grid=(NT,),
            in_specs=in_specs,
            out_specs=pl.BlockSpec((BM, H), lambda i, te: (i, 0)),
        )
        return pl.pallas_call(
            body,
            grid_spec=grid_spec,
            out_shape=jax.ShapeDtypeStruct((NP, H), jnp.float32),
            compiler_params=pltpu.CompilerParams(
                dimension_semantics=("arbitrary",)),
        )(*args)

    oa = half_call(0, None)
    og = half_call(1, oa)

    p0 = ppos[0::TOPK]
    p1 = ppos[1::TOPK]
    return _sc_combine(og, p0, p1, T)
